# baseline (speedup 1.0000x reference)
import functools

import jax
import jax.numpy as jnp
from jax import lax
from jax.experimental import pallas as pl
from jax.experimental.pallas import tpu as pltpu
from jax.experimental.pallas import tpu_sc as plsc

_VOCAB = 100000
_EMB = 64
_NEG = 200
_EPS = 1e-15
_NSUB = 16
_R = 16
_PAD = _NSUB * _R
_LN2 = 0.6931471805599453
_SQRT2 = 1.4142135623730951


def _neg_log(a):
    bits = plsc.bitcast(a, jnp.int32)
    e = (bits >> 23) - 127
    m = plsc.bitcast((bits & 0x7FFFFF) | 0x3F800000, jnp.float32)
    big = m > _SQRT2
    m = jnp.where(big, m * 0.5, m)
    e = jnp.where(big, e + 1, e)
    z = (m - 1.0) / (m + 1.0)
    z2 = z * z
    p = 1.0 + z2 * (1.0 / 3.0 + z2 * (1.0 / 5.0 + z2 * (1.0 / 7.0 + z2 * (1.0 / 9.0))))
    return -(e.astype(jnp.float32) * _LN2 + 2.0 * z * p)


def _sc_call(x_arr, signatures, nuf, neg_rand_pad):
    mesh = plsc.VectorSubcoreMesh(core_axis_name="c", subcore_axis_name="s")

    @functools.partial(
        pl.kernel,
        out_type=jax.ShapeDtypeStruct((16,), jnp.float32),
        mesh=mesh,
        compiler_params=pltpu.CompilerParams(
            needs_layout_passes=False, use_tc_tiling_on_sc=False),
        scratch_types=[
            pltpu.VMEM((_R,), jnp.int32),
            pltpu.VMEM((8,), jnp.int32),
            pltpu.VMEM((_R,), jnp.int32),
            pltpu.VMEM((_R, _EMB), jnp.float32),
            pltpu.VMEM((8, _EMB), jnp.float32),
            pltpu.VMEM((16,), jnp.float32),
            pltpu.VMEM((_NSUB, 16), jnp.float32),
            pltpu.VMEM((16,), jnp.float32),
            pltpu.VMEM_SHARED((_NSUB, 16), jnp.float32),
            pltpu.SemaphoreType.DMA,
            pltpu.SemaphoreType.DMA,
        ],
    )
    def k(x_hbm, sig_hbm, nuf_hbm, nr_hbm, out_hbm,
          myidx_v, xv, ctx_v, rows_v, tv, ybuf, red_v, outv, shared, sem1, sem2):
        c = lax.axis_index("c")
        s = lax.axis_index("s")

        @pl.when(c == 0)
        def _():
            base = s * _R
            pltpu.sync_copy(nr_hbm.at[pl.ds(base, _R)], myidx_v)
            pltpu.sync_copy(x_hbm, xv)
            tgt_cp = pltpu.async_copy(sig_hbm.at[xv], tv, sem2)
            pltpu.async_copy(nuf_hbm.at[myidx_v], ctx_v, sem1).wait()
            rows_cp = pltpu.async_copy(sig_hbm.at[ctx_v], rows_v, sem1)
            tgt_cp.wait()
            rows_cp.wait()

            iota = lax.iota(jnp.int32, 16)
            acc = jnp.zeros((16,), jnp.float32)
            for k in range(_EMB // 16):
                tvk = tv[0, pl.ds(k * 16, 16)]
                for j in range(16):
                    d = k * 16 + j
                    col = plsc.load_gather(
                        rows_v, [iota, jnp.full((16,), d, jnp.int32)])
                    acc = acc + col * tvk[j]

            a = 1.0 - 1.0 / (1.0 + jnp.exp(-acc)) + _EPS
            y = _neg_log(a)
            y = jnp.where(base + iota < _NEG, y, 0.0)
            ybuf[...] = y
            pltpu.sync_copy(ybuf, shared.at[s])
            plsc.subcore_barrier()

            @pl.when(s == 0)
            def _():
                pltpu.sync_copy(shared, red_v)
                tot = jnp.zeros((16,), jnp.float32)
                for i in range(_NSUB):
                    tot = tot + red_v[i, :]
                loss = jnp.sum(tot) * (1.0 / _NEG)
                outv[...] = jnp.full((16,), loss, jnp.float32)
                pltpu.sync_copy(outv, out_hbm)

    return k(x_arr, signatures, nuf, neg_rand_pad)


def kernel(x, signatures, negative_unit_filter, neg_rand):
    x32 = jnp.asarray(x, jnp.int32)
    x_arr = jnp.full((8,), x32, jnp.int32)
    nuf = jnp.asarray(negative_unit_filter, jnp.int32)
    nr_pad = jnp.zeros((_PAD,), jnp.int32).at[:_NEG].set(
        jnp.asarray(neg_rand, jnp.int32))
    out = _sc_call(x_arr, signatures, nuf, nr_pad)
    return (jnp.asarray(x), out[0])
